# baseline (speedup 1.0000x reference)
import functools

import jax
import jax.numpy as jnp
from jax import lax
from jax.experimental import pallas as pl
from jax.experimental.pallas import tpu as pltpu
from jax.experimental.pallas import tpu_sc as plsc

N = 38332
E = 613312
NP = 40960
EP = 622592
ER = EP // 128
ERT = ER // 16
NZ = 38400
RPT = NZ // 16
EHT = ERT // 2
LRT = NP // 32 // 128
RB = 512
GRID = NP // RB

_mesh = plsc.VectorSubcoreMesh(core_axis_name="c", subcore_axis_name="s")
_sc_params = pltpu.CompilerParams(use_tc_tiling_on_sc=False)
_sc_params_nl = pltpu.CompilerParams(use_tc_tiling_on_sc=False,
                                     needs_layout_passes=False)


def _ring(y, srcv, dstv, rows, zsh, gsem, ssem, n):
    K, G, S = 8, 6, 2
    drain = lambda semref, b: pltpu.make_async_copy(
        y.at[pl.ds(0, 128)], rows.at[b], semref).wait()
    for b in range(G):
        pltpu.async_copy(y.at[srcv.at[b]], rows.at[b], gsem)
    main = (n // K) * K

    def chunk(i, carry):
        for b in range(K):
            j = i * K + b
            drain(gsem, b)
            pltpu.async_copy(rows.at[b], zsh.at[dstv.at[j]], ssem, add=True)

            @pl.when(j >= S)
            def _():
                drain(ssem, b)

            @pl.when(j + G < main + (n - main))
            def _():
                pltpu.async_copy(y.at[srcv.at[j + G]],
                                 rows.at[(b + G) % K], gsem)
        return carry

    lax.fori_loop(0, n // K, chunk, 0)
    for j in range(main, n):
        b = j % K
        drain(gsem, b)
        pltpu.async_copy(rows.at[b], zsh.at[dstv.at[j]], ssem, add=True)
        drain(ssem, b)
    for b in range(S):
        drain(ssem, b)

@functools.partial(
    pl.kernel,
    out_type=jax.ShapeDtypeStruct((2, 2, NP, 16), jnp.float32),
    mesh=_mesh,
    compiler_params=_sc_params,
    scratch_types=[
        pltpu.VMEM((EHT, 128), jnp.int32),
        pltpu.VMEM((EHT, 128), jnp.int32),
        pltpu.VMEM((8, 128, 16), jnp.float32),
        pltpu.VMEM_SHARED((NZ, 16), jnp.float32),
        pltpu.SemaphoreType.DMA,
        pltpu.SemaphoreType.DMA,
    ],
)
def _sc_prop(y, src4, dst2, z_out, srcv, dstv, rows, zsh, gsem, ssem):
    c = lax.axis_index("c")
    s = lax.axis_index("s")
    for p in range(2):
        pltpu.sync_copy(y.at[pl.ds((2 * p + c) * NP + s * RPT, RPT)],
                        zsh.at[pl.ds(s * RPT, RPT)])
        pltpu.sync_copy(src4.at[p, c, s, pl.ds(0, EHT)], srcv)
        pltpu.sync_copy(dst2.at[s, pl.ds(0, EHT)], dstv)
        plsc.subcore_barrier()
        for h in range(2):
            _ring(y, srcv, dstv, rows, zsh, gsem, ssem, EHT)
            if h == 0:
                pltpu.sync_copy(src4.at[p, c, s, pl.ds(EHT, EHT)], srcv)
                pltpu.sync_copy(dst2.at[s, pl.ds(EHT, EHT)], dstv)
        plsc.subcore_barrier()
        pltpu.sync_copy(zsh.at[pl.ds(s * RPT, RPT)],
                        z_out.at[p, c, pl.ds(s * RPT, RPT)])


NZD = 384


@functools.partial(
    pl.kernel,
    out_type=jax.ShapeDtypeStruct((2, NZD, 128), jnp.float32),
    mesh=_mesh,
    compiler_params=_sc_params_nl,
    scratch_types=[
        pltpu.VMEM((EHT, 128), jnp.int32),
        pltpu.VMEM((NZD, 128), jnp.float32),
        pltpu.VMEM((3, 128), jnp.int32),
        pltpu.VMEM_SHARED((NZD, 128), jnp.float32),
    ],
)
def _sc_deg(dst2, zeros_d, iota3, out, dstv, degv, iotav, acc):
    c = lax.axis_index("c")
    s = lax.axis_index("s")
    pltpu.sync_copy(dst2.at[s, pl.ds(c * EHT, EHT)], dstv)
    pltpu.sync_copy(zeros_d, degv)
    pltpu.sync_copy(iota3, iotav)

    @pl.when(s == 0)
    def _():
        pltpu.sync_copy(zeros_d, acc)

    ones = jnp.full((16,), 1.0, jnp.float32)

    def row(r, carry):
        for k in range(8):
            idx = dstv[r, pl.ds(k * 16, 16)]
            plsc.addupdate_scatter(degv, [idx >> 7, idx & 127], ones)
        return carry

    lax.fori_loop(0, EHT, row, 0)
    plsc.subcore_barrier()
    for k in range(NZD // 128):
        pltpu.sync_copy(degv.at[pl.ds(k * 128, 128)], acc.at[iotav.at[k]],
                        add=True)
    plsc.subcore_barrier()
    pltpu.sync_copy(acc.at[pl.ds(s * (NZD // 16), NZD // 16)],
                    out.at[c, pl.ds(s * (NZD // 16), NZD // 16)])


@functools.partial(
    pl.kernel,
    out_type=jax.ShapeDtypeStruct((2, NZ, 16), jnp.float32),
    mesh=_mesh,
    compiler_params=_sc_params,
    scratch_types=[
        pltpu.VMEM((EHT, 128), jnp.int32),
        pltpu.VMEM((EHT, 128), jnp.int32),
        pltpu.VMEM((8, 128, 16), jnp.float32),
        pltpu.VMEM_SHARED((NZ, 16), jnp.float32),
        pltpu.SemaphoreType.DMA,
        pltpu.SemaphoreType.DMA,
    ],
)
def _sc_prop1(y16, yinit, src1, dst2, z_out, srcv, dstv, rows, zsh, gsem,
              ssem):
    c = lax.axis_index("c")
    s = lax.axis_index("s")
    pltpu.sync_copy(yinit.at[c, pl.ds(s * RPT, RPT)],
                    zsh.at[pl.ds(s * RPT, RPT)])
    pltpu.sync_copy(src1.at[s, pl.ds(c * EHT, EHT)], srcv)
    pltpu.sync_copy(dst2.at[s, pl.ds(c * EHT, EHT)], dstv)
    plsc.subcore_barrier()
    _ring(y16, srcv, dstv, rows, zsh, gsem, ssem, EHT)
    plsc.subcore_barrier()
    pltpu.sync_copy(zsh.at[pl.ds(s * RPT, RPT)],
                    z_out.at[c, pl.ds(s * RPT, RPT)])


@functools.partial(
    pl.kernel,
    out_type=[jax.ShapeDtypeStruct((NP, 64), jnp.float32),
              jax.ShapeDtypeStruct((NP, 64), jnp.float32)],
    mesh=_mesh,
    compiler_params=_sc_params,
    scratch_types=[
        pltpu.VMEM((LRT, 128), jnp.int32),
        pltpu.VMEM((LRT, 128), jnp.int32),
        pltpu.VMEM((128, 64), jnp.float32),
        pltpu.SemaphoreType.DMA,
    ],
)
def _sc_gather2(tab1, tab2, idx1, idx2, out1, out2, iv1, iv2, rows, sem):
    c = lax.axis_index("c")
    s = lax.axis_index("s")
    w = s * 2 + c
    pltpu.sync_copy(idx1.at[w], iv1)
    pltpu.sync_copy(idx2.at[w], iv2)

    def chunk(j, carry):
        pltpu.async_copy(tab1.at[iv1.at[j]], rows, sem).wait()
        pltpu.sync_copy(rows, out1.at[pl.ds(w * LRT * 128 + j * 128, 128)])
        pltpu.async_copy(tab2.at[iv2.at[j]], rows, sem).wait()
        pltpu.sync_copy(rows, out2.at[pl.ds(w * LRT * 128 + j * 128, 128)])
        return carry

    lax.fori_loop(0, LRT, chunk, 0)


def _leaky(t):
    return jnp.where(t >= 0, t, 0.01 * t)


def _tab_poi_body(emb, w, out):
    out[...] = jnp.dot(emb[...], w[...], preferred_element_type=jnp.float32)


def _tab_poi(poi_emb, w):
    return pl.pallas_call(
        _tab_poi_body,
        grid=(75,),
        in_specs=[pl.BlockSpec((512, 300), lambda i: (i, 0)),
                  pl.BlockSpec((300, 64), lambda i: (0, 0))],
        out_specs=pl.BlockSpec((512, 64), lambda i: (i, 0)),
        out_shape=jax.ShapeDtypeStruct((38400, 64), jnp.float32),
    )(poi_emb, w)


def _tab_cat(cat_emb, w):
    return pl.pallas_call(
        _tab_poi_body,
        grid=(1,),
        in_specs=[pl.BlockSpec((400, 100), lambda i: (0, 0)),
                  pl.BlockSpec((100, 64), lambda i: (0, 0))],
        out_specs=pl.BlockSpec((400, 64), lambda i: (0, 0)),
        out_shape=jax.ShapeDtypeStruct((400, 64), jnp.float32),
    )(cat_emb, w)


def _dinv_body(degT, out):
    i = pl.program_id(0)
    deg = 1.0 + degT[:, 0:1] + degT[:, 1:2]
    row = i * RB + lax.broadcasted_iota(jnp.int32, (RB, 1), 0)
    out[...] = jnp.where(row < N, lax.rsqrt(jnp.maximum(deg, 1e-6)), 0.0)


def _dinv(degT):
    return pl.pallas_call(
        _dinv_body,
        grid=(GRID,),
        in_specs=[pl.BlockSpec((RB, 2), lambda i: (i, 0))],
        out_specs=pl.BlockSpec((RB, 1), lambda i: (i, 0)),
        out_shape=jax.ShapeDtypeStruct((NP, 1), jnp.float32),
    )(degT)


def _store_quarters(out, y):
    out[0, 0] = y[:, 0:16]
    out[0, 1] = y[:, 16:32]
    out[1, 0] = y[:, 32:48]
    out[1, 1] = y[:, 48:64]


def _cat_quarters(z4):
    return jnp.concatenate([z4[0, 0], z4[0, 1], z4[1, 0], z4[1, 1]], axis=-1)


def _y0_body(pg, cg, xs, wx, dinv, out):
    xw = pg[...] + cg[...] + jnp.dot(xs[...], wx[...],
                                     preferred_element_type=jnp.float32)
    _store_quarters(out, dinv[...] * xw)


def _y0(pg, cg, xs, wx, dinv):
    return pl.pallas_call(
        _y0_body,
        grid=(GRID,),
        in_specs=[pl.BlockSpec((RB, 64), lambda i: (i, 0)),
                  pl.BlockSpec((RB, 64), lambda i: (i, 0)),
                  pl.BlockSpec((RB, 3), lambda i: (i, 0)),
                  pl.BlockSpec((3, 64), lambda i: (0, 0)),
                  pl.BlockSpec((RB, 1), lambda i: (i, 0))],
        out_specs=pl.BlockSpec((2, 2, RB, 16), lambda i: (0, 0, i, 0)),
        out_shape=jax.ShapeDtypeStruct((2, 2, NP, 16), jnp.float32),
    )(pg, cg, xs, wx, dinv)


def _layer_body(z4, dinv, b, w, out, *, residual):
    z = _cat_quarters(z4)
    t = dinv[...] * z + b[...]
    feat = _leaky(t) + t if residual else _leaky(t)
    y = dinv[...] * jnp.dot(feat, w[...], preferred_element_type=jnp.float32)
    _store_quarters(out, y)


def _layer(z4, dinv, b, w, residual):
    return pl.pallas_call(
        functools.partial(_layer_body, residual=residual),
        grid=(GRID,),
        in_specs=[pl.BlockSpec((2, 2, RB, 16), lambda i: (0, 0, i, 0)),
                  pl.BlockSpec((RB, 1), lambda i: (i, 0)),
                  pl.BlockSpec((1, 64), lambda i: (0, 0)),
                  pl.BlockSpec((64, 64), lambda i: (0, 0))],
        out_specs=pl.BlockSpec((2, 2, RB, 16), lambda i: (0, 0, i, 0)),
        out_shape=jax.ShapeDtypeStruct((2, 2, NP, 16), jnp.float32),
    )(z4, dinv, b.reshape(1, 64), w)


def _ylast_body(z4, dinv, b, w, out):
    z = _cat_quarters(z4)
    t = dinv[...] * z + b[...]
    feat = _leaky(t) + t
    y = dinv[...] * jnp.dot(feat, w[...], preferred_element_type=jnp.float32)
    out[...] = jnp.concatenate([y, jnp.zeros((RB, 15), jnp.float32)], axis=-1)


def _ylast(z4, dinv, b, w):
    return pl.pallas_call(
        _ylast_body,
        grid=(GRID,),
        in_specs=[pl.BlockSpec((2, 2, RB, 16), lambda i: (0, 0, i, 0)),
                  pl.BlockSpec((RB, 1), lambda i: (i, 0)),
                  pl.BlockSpec((1, 64), lambda i: (0, 0)),
                  pl.BlockSpec((64, 1), lambda i: (0, 0))],
        out_specs=pl.BlockSpec((RB, 16), lambda i: (i, 0)),
        out_shape=jax.ShapeDtypeStruct((NP, 16), jnp.float32),
    )(z4, dinv, b.reshape(1, 64), w)


def _flat_body(zp, dinv, b, out):
    i = pl.program_id(0)
    t = dinv[...] * (zp[0, :, 0:1] + zp[1, :, 0:1]) + b[0, 0]
    row = i * RB + lax.broadcasted_iota(jnp.int32, (RB, 1), 0)
    out[...] = jnp.where(row < N, _leaky(t), 0.0)


def _flat(zp, dinv, b):
    return pl.pallas_call(
        _flat_body,
        grid=(NZ // RB,),
        in_specs=[pl.BlockSpec((2, RB, 16), lambda i: (0, i, 0)),
                  pl.BlockSpec((RB, 1), lambda i: (i, 0)),
                  pl.BlockSpec((1, 1), lambda i: (0, 0))],
        out_specs=pl.BlockSpec((RB, 1), lambda i: (i, 0)),
        out_shape=jax.ShapeDtypeStruct((NZ, 1), jnp.float32),
    )(zp, dinv, b.reshape(1, 1))


def _fc1_body(flat, w, b, out):
    f = flat[...][:N]
    h = jnp.dot(w[...], f, preferred_element_type=jnp.float32)
    out[...] = jnp.maximum(h + b[...], 0.0)


def _fc1(flat, w, b):
    return pl.pallas_call(
        _fc1_body,
        grid=(1,),
        in_specs=[pl.BlockSpec((NZ, 1), lambda i: (0, 0)),
                  pl.BlockSpec((128, N), lambda i: (0, 0)),
                  pl.BlockSpec((128, 1), lambda i: (0, 0))],
        out_specs=pl.BlockSpec((128, 1), lambda i: (0, 0)),
        out_shape=jax.ShapeDtypeStruct((128, 1), jnp.float32),
    )(flat, w, b.reshape(128, 1))


def _fc2_body(h, w, b, out):
    o = jnp.dot(w[...], h[...], preferred_element_type=jnp.float32)
    out[...] = jnp.maximum(o + b[...], 0.0)


def _fc2(h, w, b):
    return pl.pallas_call(
        _fc2_body,
        grid=(1,),
        in_specs=[pl.BlockSpec((128, 1), lambda i: (0, 0)),
                  pl.BlockSpec((38333, 128), lambda i: (0, 0)),
                  pl.BlockSpec((38333, 1), lambda i: (0, 0))],
        out_specs=pl.BlockSpec((38333, 1), lambda i: (0, 0)),
        out_shape=jax.ShapeDtypeStruct((38333, 1), jnp.float32),
    )(h, w, b.reshape(38333, 1))


def kernel(x, edge_index, poi_emb, cat_emb, W_in, b_in, Wg0, bg0, Wg1, bg1,
           Wg2, bg2, Wg3, bg3, Wg4, bg4, W_out, b_out, Wf1, bf1, Wf2, bf2):
    f32 = jnp.float32
    poi_idx = x[:, 0].astype(jnp.int32)
    cat_idx = x[:, 1].astype(jnp.int32)
    zero_pad = jnp.zeros((NP - N,), jnp.int32)
    idx1 = jnp.concatenate([poi_idx, zero_pad]).reshape(32, LRT, 128)
    idx2 = jnp.concatenate([cat_idx, zero_pad]).reshape(32, LRT, 128)
    xs = jnp.concatenate([x[:, 2:5], jnp.zeros((NP - N, 3), f32)], axis=0)

    src = jnp.concatenate([edge_index[0], jnp.zeros((EP - E,), jnp.int32)])
    dst = jnp.concatenate([edge_index[1],
                           jnp.full((EP - E,), N, jnp.int32)])
    src4 = (src[None, :] +
            (jnp.arange(4, dtype=jnp.int32) * NP)[:, None]
            ).reshape(2, 2, 16, ERT, 128)
    dst2 = dst.reshape(16, ERT, 128)

    deg2 = _sc_deg(dst2, jnp.zeros((NZD, 128), f32),
                   jnp.arange(NZD, dtype=jnp.int32).reshape(3, 128))
    degT = deg2.reshape(2, NZD * 128)[:, :NP].T
    dinv = _dinv(degT)

    tab1 = _tab_poi(poi_emb, W_in[:300])
    tab2 = _tab_cat(cat_emb, W_in[300:400])
    pg, cg = _sc_gather2(tab1, tab2, idx1, idx2)

    y = _y0(pg, cg, xs, W_in[400:403], dinv)
    z = _sc_prop(y.reshape(4 * NP, 16), src4, dst2)
    y = _layer(z, dinv, b_in, Wg0, residual=False)
    for w, b_prev in ((Wg1, bg0), (Wg2, bg1), (Wg3, bg2), (Wg4, bg3)):
        z = _sc_prop(y.reshape(4 * NP, 16), src4, dst2)
        y = _layer(z, dinv, b_prev, w, residual=True)
    z = _sc_prop(y.reshape(4 * NP, 16), src4, dst2)
    y16 = _ylast(z, dinv, bg4, W_out)
    yinit = jnp.stack([y16[:NZ], jnp.zeros((NZ, 16), f32)])
    zfin = _sc_prop1(y16, yinit, src.reshape(16, ERT, 128), dst2)
    flat = _flat(zfin, dinv, b_out)
    h = _fc1(flat, Wf1, bf1)
    out = _fc2(h, Wf2, bf2)
    return out.reshape(38333)


# speedup vs baseline: 1.0189x; 1.0189x over previous
import functools

import jax
import jax.numpy as jnp
from jax import lax
from jax.experimental import pallas as pl
from jax.experimental.pallas import tpu as pltpu
from jax.experimental.pallas import tpu_sc as plsc

N = 38332
E = 613312
NP = 40960
EP = 622592
ER = EP // 128
ERT = ER // 16
NZ = 38400
RPT = NZ // 16
EHT = ERT // 2
LRT = NP // 32 // 128
RB = 512
GRID = NP // RB

_mesh = plsc.VectorSubcoreMesh(core_axis_name="c", subcore_axis_name="s")
_sc_params = pltpu.CompilerParams(use_tc_tiling_on_sc=False)
_sc_params_nl = pltpu.CompilerParams(use_tc_tiling_on_sc=False,
                                     needs_layout_passes=False)


def _ring(y, srcv, dstv, rows, zsh, gsem, ssem, n):
    K, G, S = 8, 7, 1
    drain = lambda semref, b: pltpu.make_async_copy(
        y.at[pl.ds(0, 128)], rows.at[b], semref).wait()
    for b in range(G):
        pltpu.async_copy(y.at[srcv.at[b]], rows.at[b], gsem)
    main = (n // K) * K

    def chunk(i, carry):
        for b in range(K):
            j = i * K + b
            drain(gsem, b)
            pltpu.async_copy(rows.at[b], zsh.at[dstv.at[j]], ssem, add=True)

            @pl.when(j >= S)
            def _():
                drain(ssem, b)

            @pl.when(j + G < main + (n - main))
            def _():
                pltpu.async_copy(y.at[srcv.at[j + G]],
                                 rows.at[(b + G) % K], gsem)
        return carry

    lax.fori_loop(0, n // K, chunk, 0)
    for j in range(main, n):
        b = j % K
        drain(gsem, b)
        pltpu.async_copy(rows.at[b], zsh.at[dstv.at[j]], ssem, add=True)
        drain(ssem, b)
    for b in range(S):
        drain(ssem, b)

@functools.partial(
    pl.kernel,
    out_type=jax.ShapeDtypeStruct((2, 2, NP, 16), jnp.float32),
    mesh=_mesh,
    compiler_params=_sc_params,
    scratch_types=[
        pltpu.VMEM((EHT, 128), jnp.int32),
        pltpu.VMEM((EHT, 128), jnp.int32),
        pltpu.VMEM((8, 128, 16), jnp.float32),
        pltpu.VMEM_SHARED((NZ, 16), jnp.float32),
        pltpu.SemaphoreType.DMA,
        pltpu.SemaphoreType.DMA,
    ],
)
def _sc_prop(y, src4, dst2, z_out, srcv, dstv, rows, zsh, gsem, ssem):
    c = lax.axis_index("c")
    s = lax.axis_index("s")
    for p in range(2):
        pltpu.sync_copy(y.at[pl.ds((2 * p + c) * NP + s * RPT, RPT)],
                        zsh.at[pl.ds(s * RPT, RPT)])
        pltpu.sync_copy(src4.at[p, c, s, pl.ds(0, EHT)], srcv)
        pltpu.sync_copy(dst2.at[s, pl.ds(0, EHT)], dstv)
        plsc.subcore_barrier()
        for h in range(2):
            _ring(y, srcv, dstv, rows, zsh, gsem, ssem, EHT)
            if h == 0:
                pltpu.sync_copy(src4.at[p, c, s, pl.ds(EHT, EHT)], srcv)
                pltpu.sync_copy(dst2.at[s, pl.ds(EHT, EHT)], dstv)
        plsc.subcore_barrier()
        pltpu.sync_copy(zsh.at[pl.ds(s * RPT, RPT)],
                        z_out.at[p, c, pl.ds(s * RPT, RPT)])


NZD = 384


@functools.partial(
    pl.kernel,
    out_type=jax.ShapeDtypeStruct((2, NZD, 128), jnp.float32),
    mesh=_mesh,
    compiler_params=_sc_params_nl,
    scratch_types=[
        pltpu.VMEM((EHT, 128), jnp.int32),
        pltpu.VMEM((NZD, 128), jnp.float32),
        pltpu.VMEM((3, 128), jnp.int32),
        pltpu.VMEM_SHARED((NZD, 128), jnp.float32),
    ],
)
def _sc_deg(dst2, zeros_d, iota3, out, dstv, degv, iotav, acc):
    c = lax.axis_index("c")
    s = lax.axis_index("s")
    pltpu.sync_copy(dst2.at[s, pl.ds(c * EHT, EHT)], dstv)
    pltpu.sync_copy(zeros_d, degv)
    pltpu.sync_copy(iota3, iotav)

    @pl.when(s == 0)
    def _():
        pltpu.sync_copy(zeros_d, acc)

    ones = jnp.full((16,), 1.0, jnp.float32)

    def row(r, carry):
        for k in range(8):
            idx = dstv[r, pl.ds(k * 16, 16)]
            plsc.addupdate_scatter(degv, [idx >> 7, idx & 127], ones)
        return carry

    lax.fori_loop(0, EHT, row, 0)
    plsc.subcore_barrier()
    for k in range(NZD // 128):
        pltpu.sync_copy(degv.at[pl.ds(k * 128, 128)], acc.at[iotav.at[k]],
                        add=True)
    plsc.subcore_barrier()
    pltpu.sync_copy(acc.at[pl.ds(s * (NZD // 16), NZD // 16)],
                    out.at[c, pl.ds(s * (NZD // 16), NZD // 16)])


@functools.partial(
    pl.kernel,
    out_type=jax.ShapeDtypeStruct((2, NZ, 16), jnp.float32),
    mesh=_mesh,
    compiler_params=_sc_params,
    scratch_types=[
        pltpu.VMEM((EHT, 128), jnp.int32),
        pltpu.VMEM((EHT, 128), jnp.int32),
        pltpu.VMEM((8, 128, 16), jnp.float32),
        pltpu.VMEM_SHARED((NZ, 16), jnp.float32),
        pltpu.SemaphoreType.DMA,
        pltpu.SemaphoreType.DMA,
    ],
)
def _sc_prop1(y16, yinit, src1, dst2, z_out, srcv, dstv, rows, zsh, gsem,
              ssem):
    c = lax.axis_index("c")
    s = lax.axis_index("s")
    pltpu.sync_copy(yinit.at[c, pl.ds(s * RPT, RPT)],
                    zsh.at[pl.ds(s * RPT, RPT)])
    pltpu.sync_copy(src1.at[s, pl.ds(c * EHT, EHT)], srcv)
    pltpu.sync_copy(dst2.at[s, pl.ds(c * EHT, EHT)], dstv)
    plsc.subcore_barrier()
    _ring(y16, srcv, dstv, rows, zsh, gsem, ssem, EHT)
    plsc.subcore_barrier()
    pltpu.sync_copy(zsh.at[pl.ds(s * RPT, RPT)],
                    z_out.at[c, pl.ds(s * RPT, RPT)])


@functools.partial(
    pl.kernel,
    out_type=[jax.ShapeDtypeStruct((NP, 64), jnp.float32),
              jax.ShapeDtypeStruct((NP, 64), jnp.float32)],
    mesh=_mesh,
    compiler_params=_sc_params,
    scratch_types=[
        pltpu.VMEM((LRT, 128), jnp.int32),
        pltpu.VMEM((LRT, 128), jnp.int32),
        pltpu.VMEM((128, 64), jnp.float32),
        pltpu.SemaphoreType.DMA,
    ],
)
def _sc_gather2(tab1, tab2, idx1, idx2, out1, out2, iv1, iv2, rows, sem):
    c = lax.axis_index("c")
    s = lax.axis_index("s")
    w = s * 2 + c
    pltpu.sync_copy(idx1.at[w], iv1)
    pltpu.sync_copy(idx2.at[w], iv2)

    def chunk(j, carry):
        pltpu.async_copy(tab1.at[iv1.at[j]], rows, sem).wait()
        pltpu.sync_copy(rows, out1.at[pl.ds(w * LRT * 128 + j * 128, 128)])
        pltpu.async_copy(tab2.at[iv2.at[j]], rows, sem).wait()
        pltpu.sync_copy(rows, out2.at[pl.ds(w * LRT * 128 + j * 128, 128)])
        return carry

    lax.fori_loop(0, LRT, chunk, 0)


def _leaky(t):
    return jnp.where(t >= 0, t, 0.01 * t)


def _tab_poi_body(emb, w, out):
    out[...] = jnp.dot(emb[...], w[...], preferred_element_type=jnp.float32)


def _tab_poi(poi_emb, w):
    return pl.pallas_call(
        _tab_poi_body,
        grid=(75,),
        in_specs=[pl.BlockSpec((512, 300), lambda i: (i, 0)),
                  pl.BlockSpec((300, 64), lambda i: (0, 0))],
        out_specs=pl.BlockSpec((512, 64), lambda i: (i, 0)),
        out_shape=jax.ShapeDtypeStruct((38400, 64), jnp.float32),
    )(poi_emb, w)


def _tab_cat(cat_emb, w):
    return pl.pallas_call(
        _tab_poi_body,
        grid=(1,),
        in_specs=[pl.BlockSpec((400, 100), lambda i: (0, 0)),
                  pl.BlockSpec((100, 64), lambda i: (0, 0))],
        out_specs=pl.BlockSpec((400, 64), lambda i: (0, 0)),
        out_shape=jax.ShapeDtypeStruct((400, 64), jnp.float32),
    )(cat_emb, w)


def _dinv_body(degT, out):
    i = pl.program_id(0)
    deg = 1.0 + degT[:, 0:1] + degT[:, 1:2]
    row = i * RB + lax.broadcasted_iota(jnp.int32, (RB, 1), 0)
    out[...] = jnp.where(row < N, lax.rsqrt(jnp.maximum(deg, 1e-6)), 0.0)


def _dinv(degT):
    return pl.pallas_call(
        _dinv_body,
        grid=(GRID,),
        in_specs=[pl.BlockSpec((RB, 2), lambda i: (i, 0))],
        out_specs=pl.BlockSpec((RB, 1), lambda i: (i, 0)),
        out_shape=jax.ShapeDtypeStruct((NP, 1), jnp.float32),
    )(degT)


def _store_quarters(out, y):
    out[0, 0] = y[:, 0:16]
    out[0, 1] = y[:, 16:32]
    out[1, 0] = y[:, 32:48]
    out[1, 1] = y[:, 48:64]


def _cat_quarters(z4):
    return jnp.concatenate([z4[0, 0], z4[0, 1], z4[1, 0], z4[1, 1]], axis=-1)


def _y0_body(pg, cg, xs, wx, dinv, out):
    xw = pg[...] + cg[...] + jnp.dot(xs[...], wx[...],
                                     preferred_element_type=jnp.float32)
    _store_quarters(out, dinv[...] * xw)


def _y0(pg, cg, xs, wx, dinv):
    return pl.pallas_call(
        _y0_body,
        grid=(GRID,),
        in_specs=[pl.BlockSpec((RB, 64), lambda i: (i, 0)),
                  pl.BlockSpec((RB, 64), lambda i: (i, 0)),
                  pl.BlockSpec((RB, 3), lambda i: (i, 0)),
                  pl.BlockSpec((3, 64), lambda i: (0, 0)),
                  pl.BlockSpec((RB, 1), lambda i: (i, 0))],
        out_specs=pl.BlockSpec((2, 2, RB, 16), lambda i: (0, 0, i, 0)),
        out_shape=jax.ShapeDtypeStruct((2, 2, NP, 16), jnp.float32),
    )(pg, cg, xs, wx, dinv)


def _layer_body(z4, dinv, b, w, out, *, residual):
    z = _cat_quarters(z4)
    t = dinv[...] * z + b[...]
    feat = _leaky(t) + t if residual else _leaky(t)
    y = dinv[...] * jnp.dot(feat, w[...], preferred_element_type=jnp.float32)
    _store_quarters(out, y)


def _layer(z4, dinv, b, w, residual):
    return pl.pallas_call(
        functools.partial(_layer_body, residual=residual),
        grid=(GRID,),
        in_specs=[pl.BlockSpec((2, 2, RB, 16), lambda i: (0, 0, i, 0)),
                  pl.BlockSpec((RB, 1), lambda i: (i, 0)),
                  pl.BlockSpec((1, 64), lambda i: (0, 0)),
                  pl.BlockSpec((64, 64), lambda i: (0, 0))],
        out_specs=pl.BlockSpec((2, 2, RB, 16), lambda i: (0, 0, i, 0)),
        out_shape=jax.ShapeDtypeStruct((2, 2, NP, 16), jnp.float32),
    )(z4, dinv, b.reshape(1, 64), w)


def _ylast_body(z4, dinv, b, w, out):
    z = _cat_quarters(z4)
    t = dinv[...] * z + b[...]
    feat = _leaky(t) + t
    y = dinv[...] * jnp.dot(feat, w[...], preferred_element_type=jnp.float32)
    out[...] = jnp.concatenate([y, jnp.zeros((RB, 15), jnp.float32)], axis=-1)


def _ylast(z4, dinv, b, w):
    return pl.pallas_call(
        _ylast_body,
        grid=(GRID,),
        in_specs=[pl.BlockSpec((2, 2, RB, 16), lambda i: (0, 0, i, 0)),
                  pl.BlockSpec((RB, 1), lambda i: (i, 0)),
                  pl.BlockSpec((1, 64), lambda i: (0, 0)),
                  pl.BlockSpec((64, 1), lambda i: (0, 0))],
        out_specs=pl.BlockSpec((RB, 16), lambda i: (i, 0)),
        out_shape=jax.ShapeDtypeStruct((NP, 16), jnp.float32),
    )(z4, dinv, b.reshape(1, 64), w)


def _flat_body(zp, dinv, b, out):
    i = pl.program_id(0)
    t = dinv[...] * (zp[0, :, 0:1] + zp[1, :, 0:1]) + b[0, 0]
    row = i * RB + lax.broadcasted_iota(jnp.int32, (RB, 1), 0)
    out[...] = jnp.where(row < N, _leaky(t), 0.0)


def _flat(zp, dinv, b):
    return pl.pallas_call(
        _flat_body,
        grid=(NZ // RB,),
        in_specs=[pl.BlockSpec((2, RB, 16), lambda i: (0, i, 0)),
                  pl.BlockSpec((RB, 1), lambda i: (i, 0)),
                  pl.BlockSpec((1, 1), lambda i: (0, 0))],
        out_specs=pl.BlockSpec((RB, 1), lambda i: (i, 0)),
        out_shape=jax.ShapeDtypeStruct((NZ, 1), jnp.float32),
    )(zp, dinv, b.reshape(1, 1))


def _fc1_body(flat, w, b, out):
    f = flat[...][:N]
    h = jnp.dot(w[...], f, preferred_element_type=jnp.float32)
    out[...] = jnp.maximum(h + b[...], 0.0)


def _fc1(flat, w, b):
    return pl.pallas_call(
        _fc1_body,
        grid=(1,),
        in_specs=[pl.BlockSpec((NZ, 1), lambda i: (0, 0)),
                  pl.BlockSpec((128, N), lambda i: (0, 0)),
                  pl.BlockSpec((128, 1), lambda i: (0, 0))],
        out_specs=pl.BlockSpec((128, 1), lambda i: (0, 0)),
        out_shape=jax.ShapeDtypeStruct((128, 1), jnp.float32),
    )(flat, w, b.reshape(128, 1))


def _fc2_body(h, w, b, out):
    o = jnp.dot(w[...], h[...], preferred_element_type=jnp.float32)
    out[...] = jnp.maximum(o + b[...], 0.0)


def _fc2(h, w, b):
    return pl.pallas_call(
        _fc2_body,
        grid=(1,),
        in_specs=[pl.BlockSpec((128, 1), lambda i: (0, 0)),
                  pl.BlockSpec((38333, 128), lambda i: (0, 0)),
                  pl.BlockSpec((38333, 1), lambda i: (0, 0))],
        out_specs=pl.BlockSpec((38333, 1), lambda i: (0, 0)),
        out_shape=jax.ShapeDtypeStruct((38333, 1), jnp.float32),
    )(h, w, b.reshape(38333, 1))


def kernel(x, edge_index, poi_emb, cat_emb, W_in, b_in, Wg0, bg0, Wg1, bg1,
           Wg2, bg2, Wg3, bg3, Wg4, bg4, W_out, b_out, Wf1, bf1, Wf2, bf2):
    f32 = jnp.float32
    poi_idx = x[:, 0].astype(jnp.int32)
    cat_idx = x[:, 1].astype(jnp.int32)
    zero_pad = jnp.zeros((NP - N,), jnp.int32)
    idx1 = jnp.concatenate([poi_idx, zero_pad]).reshape(32, LRT, 128)
    idx2 = jnp.concatenate([cat_idx, zero_pad]).reshape(32, LRT, 128)
    xs = jnp.concatenate([x[:, 2:5], jnp.zeros((NP - N, 3), f32)], axis=0)

    src = jnp.concatenate([edge_index[0], jnp.zeros((EP - E,), jnp.int32)])
    dst = jnp.concatenate([edge_index[1],
                           jnp.full((EP - E,), N, jnp.int32)])
    src4 = (src[None, :] +
            (jnp.arange(4, dtype=jnp.int32) * NP)[:, None]
            ).reshape(2, 2, 16, ERT, 128)
    dst2 = dst.reshape(16, ERT, 128)

    deg2 = _sc_deg(dst2, jnp.zeros((NZD, 128), f32),
                   jnp.arange(NZD, dtype=jnp.int32).reshape(3, 128))
    degT = deg2.reshape(2, NZD * 128)[:, :NP].T
    dinv = _dinv(degT)

    tab1 = _tab_poi(poi_emb, W_in[:300])
    tab2 = _tab_cat(cat_emb, W_in[300:400])
    pg, cg = _sc_gather2(tab1, tab2, idx1, idx2)

    y = _y0(pg, cg, xs, W_in[400:403], dinv)
    z = _sc_prop(y.reshape(4 * NP, 16), src4, dst2)
    y = _layer(z, dinv, b_in, Wg0, residual=False)
    for w, b_prev in ((Wg1, bg0), (Wg2, bg1), (Wg3, bg2), (Wg4, bg3)):
        z = _sc_prop(y.reshape(4 * NP, 16), src4, dst2)
        y = _layer(z, dinv, b_prev, w, residual=True)
    z = _sc_prop(y.reshape(4 * NP, 16), src4, dst2)
    y16 = _ylast(z, dinv, bg4, W_out)
    yinit = jnp.stack([y16[:NZ], jnp.zeros((NZ, 16), f32)])
    zfin = _sc_prop1(y16, yinit, src.reshape(16, ERT, 128), dst2)
    flat = _flat(zfin, dinv, b_out)
    h = _fc1(flat, Wf1, bf1)
    out = _fc2(h, Wf2, bf2)
    return out.reshape(38333)
